# baseline (device time: 145796 ns/iter reference)
import jax
import jax.numpy as jnp
from jax import lax
from jax.experimental import pallas as pl
from jax.experimental.pallas import tpu as pltpu

N_DEV = 32
N_TOK = 512
D_MODEL = 256
D_OUT = 512
E_LOCAL = 4
CAP = 3
ROWS = N_TOK // N_DEV


def kernel(x, router_W, route_idx, expert_W):
    del router_W

    def body(x_ref, idx_ref, w_ref, out_ref, comm_ref, send_sems, recv_sems):
        my = lax.axis_index("i")
        left = lax.rem(my + N_DEV - 1, N_DEV)
        right = lax.rem(my + 1, N_DEV)

        barrier = pltpu.get_barrier_semaphore()
        for nbr in (left, right):
            pl.semaphore_signal(
                barrier, inc=1, device_id=(nbr,),
                device_id_type=pl.DeviceIdType.MESH,
            )
        pl.semaphore_wait(barrier, 2)

        route = idx_ref[:, 0:1]
        ge = lax.broadcasted_iota(jnp.int32, (N_TOK, E_LOCAL), 1) + E_LOCAL * my
        onehot = (route == ge).astype(jnp.float32)
        ti = lax.broadcasted_iota(jnp.int32, (N_TOK, N_TOK), 0)
        tj = lax.broadcasted_iota(jnp.int32, (N_TOK, N_TOK), 1)
        lower = (tj < ti).astype(jnp.float32)
        ranks = jnp.dot(lower, onehot, preferred_element_type=jnp.float32)
        keep = onehot * (ranks < CAP).astype(jnp.float32)

        xb = x_ref[:, :].astype(jnp.bfloat16)
        acc = jnp.zeros((N_TOK, D_OUT), jnp.float32)
        for e in range(E_LOCAL):
            xm = xb * keep[:, e : e + 1].astype(jnp.bfloat16)
            acc = acc + jnp.dot(
                xm,
                w_ref[e].astype(jnp.bfloat16),
                preferred_element_type=jnp.float32,
            )
        out_ref[:, :] = acc

        for s in range(N_DEV - 1):
            slot = s % 2
            send_c = lax.rem(my - s + N_DEV, N_DEV)
            recv_c = lax.rem(my - 1 - s + 2 * N_DEV, N_DEV)
            rdma = pltpu.make_async_remote_copy(
                src_ref=out_ref.at[pl.ds(send_c * ROWS, ROWS), :],
                dst_ref=comm_ref.at[slot],
                send_sem=send_sems.at[slot],
                recv_sem=recv_sems.at[slot],
                device_id=(right,),
                device_id_type=pl.DeviceIdType.MESH,
            )
            rdma.start()
            rdma.wait()
            rs = recv_c * ROWS
            out_ref[pl.ds(rs, ROWS), :] = (
                out_ref[pl.ds(rs, ROWS), :] + comm_ref[slot]
            )

        for s in range(N_DEV - 1):
            slot = (s + N_DEV - 1) % 2
            send_c = lax.rem(my + 1 - s + 2 * N_DEV, N_DEV)
            recv_c = lax.rem(my - s + 2 * N_DEV, N_DEV)
            rdma = pltpu.make_async_remote_copy(
                src_ref=out_ref.at[pl.ds(send_c * ROWS, ROWS), :],
                dst_ref=comm_ref.at[slot],
                send_sem=send_sems.at[slot],
                recv_sem=recv_sems.at[slot],
                device_id=(right,),
                device_id_type=pl.DeviceIdType.MESH,
            )
            rdma.start()
            rdma.wait()
            out_ref[pl.ds(recv_c * ROWS, ROWS), :] = comm_ref[slot]

    return pl.pallas_call(
        body,
        out_shape=jax.ShapeDtypeStruct((N_TOK, D_OUT), jnp.float32),
        in_specs=[
            pl.BlockSpec(memory_space=pltpu.VMEM),
            pl.BlockSpec(memory_space=pltpu.VMEM),
            pl.BlockSpec(memory_space=pltpu.VMEM),
        ],
        out_specs=pl.BlockSpec(memory_space=pltpu.VMEM),
        scratch_shapes=[
            pltpu.VMEM((2, ROWS, D_OUT), jnp.float32),
            pltpu.SemaphoreType.DMA((2,)),
            pltpu.SemaphoreType.DMA((2,)),
        ],
        compiler_params=pltpu.CompilerParams(collective_id=0),
    )(x, route_idx, expert_W)


# device time: 62771 ns/iter; 2.3227x vs baseline; 2.3227x over previous
import jax
import jax.numpy as jnp
from jax import lax
from jax.experimental import pallas as pl
from jax.experimental.pallas import tpu as pltpu

N_DEV = 32
N_TOK = 512
D_MODEL = 256
D_OUT = 512
E_LOCAL = 4
CAP = 3
ROWS = N_TOK // N_DEV


def kernel(x, router_W, route_idx, expert_W):
    del router_W

    def body(x_ref, idx_ref, w_ref, out_ref, comm_ref, send_sems, recv_sems):
        my = lax.axis_index("i")

        barrier = pltpu.get_barrier_semaphore()
        for b in (1, 2, 4, 8, 16):
            pl.semaphore_signal(
                barrier, inc=1, device_id=(my ^ b,),
                device_id_type=pl.DeviceIdType.MESH,
            )
        pl.semaphore_wait(barrier, 5)

        route = idx_ref[:, 0:1]
        ge = lax.broadcasted_iota(jnp.int32, (N_TOK, E_LOCAL), 1) + E_LOCAL * my
        onehot = (route == ge).astype(jnp.float32)
        ti = lax.broadcasted_iota(jnp.int32, (N_TOK, N_TOK), 0)
        tj = lax.broadcasted_iota(jnp.int32, (N_TOK, N_TOK), 1)
        lower = (tj < ti).astype(jnp.float32)
        ranks = jnp.dot(lower, onehot, preferred_element_type=jnp.float32)
        keep = onehot * (ranks < CAP).astype(jnp.float32)

        xb = x_ref[:, :].astype(jnp.bfloat16)
        acc = jnp.zeros((N_TOK, D_OUT), jnp.float32)
        for e in range(E_LOCAL):
            xm = xb * keep[:, e : e + 1].astype(jnp.bfloat16)
            acc = acc + jnp.dot(
                xm,
                w_ref[e].astype(jnp.bfloat16),
                preferred_element_type=jnp.float32,
            )
        out_ref[:, :] = acc

        rs_off = [0, 256, 384, 448, 480]
        ag_off = [496, 512, 544, 608, 736]

        for s in range(5):
            half = 16 >> s
            rows_h = half * ROWS
            partner = my ^ half
            recv_lo = (my // half) * half
            send_lo = recv_lo ^ half
            rdma = pltpu.make_async_remote_copy(
                src_ref=out_ref.at[pl.ds(send_lo * ROWS, rows_h), :],
                dst_ref=comm_ref.at[pl.ds(rs_off[s], rows_h), :],
                send_sem=send_sems.at[s],
                recv_sem=recv_sems.at[s],
                device_id=(partner,),
                device_id_type=pl.DeviceIdType.MESH,
            )
            rdma.start()
            rdma.wait()
            rs = recv_lo * ROWS
            out_ref[pl.ds(rs, rows_h), :] = (
                out_ref[pl.ds(rs, rows_h), :]
                + comm_ref[pl.ds(rs_off[s], rows_h), :]
            )

        for s in range(5):
            n_c = 1 << s
            rows_n = n_c * ROWS
            partner = my ^ n_c
            cur_lo = (my // n_c) * n_c
            partner_lo = cur_lo ^ n_c
            rdma = pltpu.make_async_remote_copy(
                src_ref=out_ref.at[pl.ds(cur_lo * ROWS, rows_n), :],
                dst_ref=comm_ref.at[pl.ds(ag_off[s], rows_n), :],
                send_sem=send_sems.at[5 + s],
                recv_sem=recv_sems.at[5 + s],
                device_id=(partner,),
                device_id_type=pl.DeviceIdType.MESH,
            )
            rdma.start()
            rdma.wait()
            out_ref[pl.ds(partner_lo * ROWS, rows_n), :] = comm_ref[
                pl.ds(ag_off[s], rows_n), :
            ]

    return pl.pallas_call(
        body,
        out_shape=jax.ShapeDtypeStruct((N_TOK, D_OUT), jnp.float32),
        in_specs=[
            pl.BlockSpec(memory_space=pltpu.VMEM),
            pl.BlockSpec(memory_space=pltpu.VMEM),
            pl.BlockSpec(memory_space=pltpu.VMEM),
        ],
        out_specs=pl.BlockSpec(memory_space=pltpu.VMEM),
        scratch_shapes=[
            pltpu.VMEM((992, D_OUT), jnp.float32),
            pltpu.SemaphoreType.DMA((10,)),
            pltpu.SemaphoreType.DMA((10,)),
        ],
        compiler_params=pltpu.CompilerParams(collective_id=0),
    )(x, route_idx, expert_W)


# device time: 30943 ns/iter; 4.7118x vs baseline; 2.0286x over previous
import jax
import jax.numpy as jnp
from jax import lax
from jax.experimental import pallas as pl
from jax.experimental.pallas import tpu as pltpu

N_DEV = 32
N_TOK = 512
D_MODEL = 256
D_OUT = 512
E_LOCAL = 4
CAP = 3
BLK = 16


def kernel(x, router_W, route_idx, expert_W):
    del router_W

    def body(x_ref, idx_ref, w_ref, out_ref, gath_ref, send_sems, recv_sems):
        my = lax.axis_index("i")

        barrier = pltpu.get_barrier_semaphore()
        for b in (1, 2, 4, 8, 16):
            pl.semaphore_signal(
                barrier, inc=1, device_id=(my ^ b,),
                device_id_type=pl.DeviceIdType.MESH,
            )
        pl.semaphore_wait(barrier, 5)

        route = idx_ref[:, 0:1]
        eall = lax.broadcasted_iota(jnp.int32, (N_TOK, 128), 1)
        onehot_all = (route == eall).astype(jnp.float32)
        ti = lax.broadcasted_iota(jnp.int32, (N_TOK, N_TOK), 0)
        tj = lax.broadcasted_iota(jnp.int32, (N_TOK, N_TOK), 1)
        lower = (tj < ti).astype(jnp.float32)
        ranks_all = jnp.dot(lower, onehot_all, preferred_element_type=jnp.float32)
        rank_f = jnp.sum(onehot_all * ranks_all, axis=1, keepdims=True)
        keep = rank_f < CAP
        rank = rank_f.astype(jnp.int32)
        col = (route // E_LOCAL) * BLK + (route % E_LOCAL) * CAP + rank

        ge = lax.broadcasted_iota(jnp.int32, (N_TOK, E_LOCAL), 1) + E_LOCAL * my
        keep_local = ((route == ge) & keep).astype(jnp.float32)
        xb = x_ref[:, :].astype(jnp.bfloat16)
        acc = jnp.zeros((N_TOK, D_OUT), jnp.float32)
        for e in range(E_LOCAL):
            xm = xb * keep_local[:, e : e + 1].astype(jnp.bfloat16)
            acc = acc + jnp.dot(
                xm,
                w_ref[e].astype(jnp.bfloat16),
                preferred_element_type=jnp.float32,
            )

        sl16 = lax.broadcasted_iota(jnp.int32, (N_TOK, BLK), 1)
        pmy = ((col == sl16 + BLK * my) & keep).astype(jnp.float32)
        block = lax.dot_general(
            pmy, acc, (((0,), (0,)), ((), ())),
            preferred_element_type=jnp.float32,
        )
        gath_ref[pl.ds(BLK * my, BLK), :] = block.astype(jnp.bfloat16)

        for s in range(5):
            n_d = 1 << s
            rows = n_d * BLK
            partner = my ^ n_d
            cur_lo = (my // n_d) * n_d
            rdma = pltpu.make_async_remote_copy(
                src_ref=gath_ref.at[pl.ds(cur_lo * BLK, rows), :],
                dst_ref=gath_ref.at[pl.ds(cur_lo * BLK, rows), :],
                send_sem=send_sems.at[s],
                recv_sem=recv_sems.at[s],
                device_id=(partner,),
                device_id_type=pl.DeviceIdType.MESH,
            )
            rdma.start()
            rdma.wait()

        jall = lax.broadcasted_iota(jnp.int32, (N_TOK, N_DEV * BLK), 1)
        p_full = ((col == jall) & keep).astype(jnp.bfloat16)
        out_ref[:, :] = jnp.dot(
            p_full, gath_ref[:, :], preferred_element_type=jnp.float32
        )

    return pl.pallas_call(
        body,
        out_shape=jax.ShapeDtypeStruct((N_TOK, D_OUT), jnp.float32),
        in_specs=[
            pl.BlockSpec(memory_space=pltpu.VMEM),
            pl.BlockSpec(memory_space=pltpu.VMEM),
            pl.BlockSpec(memory_space=pltpu.VMEM),
        ],
        out_specs=pl.BlockSpec(memory_space=pltpu.VMEM),
        scratch_shapes=[
            pltpu.VMEM((N_DEV * BLK, D_OUT), jnp.bfloat16),
            pltpu.SemaphoreType.DMA((5,)),
            pltpu.SemaphoreType.DMA((5,)),
        ],
        compiler_params=pltpu.CompilerParams(collective_id=0),
    )(x, route_idx, expert_W)


# device time: 20839 ns/iter; 6.9963x vs baseline; 1.4849x over previous
import os

import jax
import jax.numpy as jnp
from jax import lax
from jax.experimental import pallas as pl
from jax.experimental.pallas import tpu as pltpu

N_DEV = 32
N_TOK = 512
D_MODEL = 256
D_OUT = 512
E_LOCAL = 4
CAP = 3
BLK = 16


def kernel(x, router_W, route_idx, expert_W):
    del router_W

    def body(x_ref, idx_ref, w_ref, out_ref, gath_ref, send_sems, recv_sems):
        my = lax.axis_index("i")

        barrier = pltpu.get_barrier_semaphore()
        for k in range(1, N_DEV):
            pl.semaphore_signal(
                barrier, inc=1, device_id=(lax.rem(my + k, N_DEV),),
                device_id_type=pl.DeviceIdType.MESH,
            )

        route = idx_ref[:, 0:1]
        eall = lax.broadcasted_iota(jnp.int32, (N_TOK, 128), 1)
        onehot = route == eall
        ti = lax.broadcasted_iota(jnp.int32, (N_TOK, N_TOK), 0)
        tj = lax.broadcasted_iota(jnp.int32, (N_TOK, N_TOK), 1)
        lower = (tj < ti).astype(jnp.bfloat16)
        ranks_all = jnp.dot(
            lower, onehot.astype(jnp.bfloat16),
            preferred_element_type=jnp.float32,
        )
        rank_f = jnp.sum(
            jnp.where(onehot, ranks_all, 0.0), axis=1, keepdims=True
        )
        keep = rank_f < CAP
        rank = rank_f.astype(jnp.int32)
        col = (route // E_LOCAL) * BLK + (route % E_LOCAL) * CAP + rank

        sl16 = lax.broadcasted_iota(jnp.int32, (N_TOK, BLK), 1)
        pmy = ((col == sl16 + BLK * my) & keep).astype(jnp.bfloat16)
        xb = x_ref[:, :]
        xg = lax.dot_general(
            pmy, xb, (((0,), (0,)), ((), ())),
            preferred_element_type=jnp.float32,
        ).astype(jnp.bfloat16)
        sl_col = lax.broadcasted_iota(jnp.int32, (BLK, 1), 0)
        block = jnp.zeros((BLK, D_OUT), jnp.float32)
        for e in range(E_LOCAL):
            rowmask = (
                (sl_col >= e * CAP) & (sl_col < (e + 1) * CAP)
            ).astype(jnp.bfloat16)
            block = block + jnp.dot(
                xg * rowmask,
                w_ref[e],
                preferred_element_type=jnp.float32,
            )
        gath_ref[pl.ds(BLK * my, BLK), :] = block.astype(jnp.bfloat16)

        jall = lax.broadcasted_iota(jnp.int32, (N_TOK, N_DEV * BLK), 1)
        p_full = ((col == jall) & keep).astype(jnp.bfloat16)

        pl.semaphore_wait(barrier, N_DEV - 1)

        rdmas = []
        n_peers = 0 if os.environ.get("NO_COMM") else N_DEV - 1
        for k in range(1, n_peers + 1):
            peer = lax.rem(my + k, N_DEV)
            rdma = pltpu.make_async_remote_copy(
                src_ref=gath_ref.at[pl.ds(BLK * my, BLK), :],
                dst_ref=gath_ref.at[pl.ds(BLK * my, BLK), :],
                send_sem=send_sems.at[k - 1],
                recv_sem=recv_sems.at[k - 1],
                device_id=(peer,),
                device_id_type=pl.DeviceIdType.MESH,
            )
            rdma.start()
            rdmas.append(rdma)
        for rdma in rdmas:
            rdma.wait()

        out_ref[:, :] = jnp.dot(
            p_full, gath_ref[:, :], preferred_element_type=jnp.float32
        )

    return pl.pallas_call(
        body,
        out_shape=jax.ShapeDtypeStruct((N_TOK, D_OUT), jnp.float32),
        in_specs=[
            pl.BlockSpec(memory_space=pltpu.VMEM),
            pl.BlockSpec(memory_space=pltpu.VMEM),
            pl.BlockSpec(memory_space=pltpu.VMEM),
        ],
        out_specs=pl.BlockSpec(memory_space=pltpu.VMEM),
        scratch_shapes=[
            pltpu.VMEM((N_DEV * BLK, D_OUT), jnp.bfloat16),
            pltpu.SemaphoreType.DMA((N_DEV - 1,)),
            pltpu.SemaphoreType.DMA((N_DEV - 1,)),
        ],
        compiler_params=pltpu.CompilerParams(collective_id=0),
    )(x.astype(jnp.bfloat16), route_idx, expert_W.astype(jnp.bfloat16))
